# Initial kernel scaffold; baseline (speedup 1.0000x reference)
import math

import jax
import jax.numpy as jnp
from jax import lax
from jax.experimental import pallas as pl
from jax.experimental.pallas import tpu as pltpu
from jax.experimental.pallas import tpu_sc as plsc

N = 10000
E = 320000
IN_DIM = 128
OUT_DIM = 128
H = 8
DK = OUT_DIM // H

NC = 2
NS = 16
NW = NC * NS
CHUNK = 128
N_PAD = 10240
E_PER_W = ((E + NW * CHUNK - 1) // (NW * CHUNK)) * CHUNK
E_PAD = E_PER_W * NW
N_CHUNKS = E_PER_W // CHUNK
ROWS_PER_S = N_PAD // NS
ZCHUNKS = ROWS_PER_S // CHUNK


def _qkv_body(h_ref, wq_ref, wk_ref, wv_ref, bq_ref, bk_ref, bv_ref,
              ra_ref, rm_ref, pri_ref, q_ref, kv_ref):
    hb = h_ref[...]
    q = jnp.dot(hb, wq_ref[...], preferred_element_type=jnp.float32) + bq_ref[...]
    q_ref[...] = q * pri_ref[...]
    k = jnp.dot(hb, wk_ref[...], preferred_element_type=jnp.float32) + bk_ref[...]
    k2 = jnp.dot(k, ra_ref[...], preferred_element_type=jnp.float32)
    v = jnp.dot(hb, wv_ref[...], preferred_element_type=jnp.float32) + bv_ref[...]
    v2 = jnp.dot(v, rm_ref[...], preferred_element_type=jnp.float32)
    kv_ref[...] = jnp.concatenate([k2, v2], axis=1)


def _qkv(h_pad, WqT, WkT, WvT, bq, bk, bv, RA, RM, pri):
    blk = 512
    grid = N_PAD // blk
    full = lambda shape: pl.BlockSpec(shape, lambda i: (0, 0))
    return pl.pallas_call(
        _qkv_body,
        grid=(grid,),
        in_specs=[
            pl.BlockSpec((blk, IN_DIM), lambda i: (i, 0)),
            full((IN_DIM, OUT_DIM)), full((IN_DIM, OUT_DIM)), full((IN_DIM, OUT_DIM)),
            full((1, OUT_DIM)), full((1, OUT_DIM)), full((1, OUT_DIM)),
            full((OUT_DIM, OUT_DIM)), full((OUT_DIM, OUT_DIM)),
            full((1, OUT_DIM)),
        ],
        out_specs=[
            pl.BlockSpec((blk, OUT_DIM), lambda i: (i, 0)),
            pl.BlockSpec((blk, 2 * OUT_DIM), lambda i: (i, 0)),
        ],
        out_shape=[
            jax.ShapeDtypeStruct((N_PAD, OUT_DIM), jnp.float32),
            jax.ShapeDtypeStruct((N_PAD, 2 * OUT_DIM), jnp.float32),
        ],
    )(h_pad, WqT, WkT, WvT, bq, bk, bv, RA, RM, pri)


def _sc_body(q_hbm, kv_hbm, src_hbm, dst_hbm, agg_hbm, den_hbm,
             sidx, didx, qbuf, kvbuf, wvbuf, tbuf, agg_sh, den_sh,
             sem_q, sem_kv):
    c = lax.axis_index("c")
    s = lax.axis_index("s")
    wid = s * NC + c

    headmask = jnp.where(lax.iota(jnp.int32, 16) < H, 1.0, 0.0)

    def zrow(i, _):
        for j in range(OUT_DIM // 16):
            wvbuf[i, pl.ds(j * 16, 16)] = jnp.zeros((16,), jnp.float32)
        tbuf[i, :] = jnp.zeros((16,), jnp.float32)
        return 0
    lax.fori_loop(0, CHUNK, zrow, 0)
    for r in range(ZCHUNKS):
        base = s * ROWS_PER_S + r * CHUNK
        pltpu.sync_copy(wvbuf, agg_sh.at[pl.ds(base, CHUNK)])
        pltpu.sync_copy(tbuf, den_sh.at[pl.ds(base, CHUNK)])
    plsc.subcore_barrier()

    def chunk_body(g, _):
        ebase = wid * E_PER_W + g * CHUNK
        pltpu.sync_copy(src_hbm.at[pl.ds(ebase, CHUNK)], sidx)
        pltpu.sync_copy(dst_hbm.at[pl.ds(ebase, CHUNK)], didx)
        cp_q = pltpu.async_copy(q_hbm.at[didx], qbuf, sem_q)
        cp_kv = pltpu.async_copy(kv_hbm.at[sidx], kvbuf, sem_kv)
        cp_q.wait()
        cp_kv.wait()

        def dot_body(e, _):
            for hh in range(H):
                qv = qbuf[e, pl.ds(hh * DK, 16)]
                kv = kvbuf[e, pl.ds(hh * DK, 16)]
                tbuf[e, hh] = jnp.sum(qv * kv)
            return 0
        lax.fori_loop(0, CHUNK, dot_body, 0)

        def exp_body(e, _):
            tbuf[e, :] = jnp.exp(tbuf[e, :]) * headmask
            return 0
        lax.fori_loop(0, CHUNK, exp_body, 0)

        def w_body(e, _):
            for hh in range(H):
                sc = tbuf[e, hh]
                vv = kvbuf[e, pl.ds(OUT_DIM + hh * DK, 16)]
                wvbuf[e, pl.ds(hh * DK, 16)] = vv * sc
            return 0
        lax.fori_loop(0, CHUNK, w_body, 0)

        pltpu.sync_copy(wvbuf, agg_sh.at[didx], add=True)
        pltpu.sync_copy(tbuf, den_sh.at[didx], add=True)
        return 0
    lax.fori_loop(0, N_CHUNKS, chunk_body, 0)

    plsc.subcore_barrier()
    rbase = s * ROWS_PER_S
    pltpu.sync_copy(agg_sh.at[pl.ds(rbase, ROWS_PER_S)],
                    agg_hbm.at[c, pl.ds(rbase, ROWS_PER_S)])
    pltpu.sync_copy(den_sh.at[pl.ds(rbase, ROWS_PER_S)],
                    den_hbm.at[c, pl.ds(rbase, ROWS_PER_S)])


def _sc_edges(q_tab, kv_tab, src_pad, dst_pad):
    mesh = plsc.VectorSubcoreMesh(core_axis_name="c", subcore_axis_name="s",
                                  num_cores=NC, num_subcores=NS)
    kern = pl.kernel(
        _sc_body,
        out_type=[
            jax.ShapeDtypeStruct((NC, N_PAD, OUT_DIM), jnp.float32),
            jax.ShapeDtypeStruct((NC, N_PAD, 16), jnp.float32),
        ],
        mesh=mesh,
        scratch_types=[
            pltpu.VMEM((CHUNK,), jnp.int32),
            pltpu.VMEM((CHUNK,), jnp.int32),
            pltpu.VMEM((CHUNK, OUT_DIM), jnp.float32),
            pltpu.VMEM((CHUNK, 2 * OUT_DIM), jnp.float32),
            pltpu.VMEM((CHUNK, OUT_DIM), jnp.float32),
            pltpu.VMEM((CHUNK, 16), jnp.float32),
            pltpu.VMEM_SHARED((N_PAD, OUT_DIM), jnp.float32),
            pltpu.VMEM_SHARED((N_PAD, 16), jnp.float32),
            pltpu.SemaphoreType.DMA,
            pltpu.SemaphoreType.DMA,
        ],
    )
    return kern(q_tab, kv_tab, src_pad, dst_pad)


def _out_body(agg_ref, den_ref, h_ref, e16_ref, w2_ref, cb_ref, out_ref):
    a = agg_ref[0] + agg_ref[1]
    d = den_ref[0] + den_ref[1]
    dexp = jnp.dot(d, e16_ref[...], preferred_element_type=jnp.float32)
    x = a / (dexp + 1e-16)
    t = jnp.dot(x, w2_ref[...], preferred_element_type=jnp.float32)
    out_ref[...] = t + cb_ref[0, 0] * h_ref[...] + cb_ref[1:2, 1:129]


def _out(agg, den, h, E16, W2, cb):
    blk = 1000
    grid = N // blk
    full = lambda shape: pl.BlockSpec(shape, lambda i: tuple(0 for _ in shape))
    return pl.pallas_call(
        _out_body,
        grid=(grid,),
        in_specs=[
            pl.BlockSpec((NC, blk, OUT_DIM), lambda i: (0, i, 0)),
            pl.BlockSpec((NC, blk, 16), lambda i: (0, i, 0)),
            pl.BlockSpec((blk, IN_DIM), lambda i: (i, 0)),
            full((16, OUT_DIM)),
            full((OUT_DIM, OUT_DIM)),
            full((2, 130)),
        ],
        out_specs=pl.BlockSpec((blk, OUT_DIM), lambda i: (i, 0)),
        out_shape=jax.ShapeDtypeStruct((N, OUT_DIM), jnp.float32),
    )(agg, den, h, E16, W2, cb)


@jax.jit
def kernel(h, edge_index, Wk, bk, Wq, bq, Wv, bv, Wa, ba, rel_att, rel_msg,
           rel_pri, skip):
    f32 = jnp.float32
    WqT = Wq.T
    WkT = Wk.T
    WvT = Wv.T
    RA = jnp.zeros((OUT_DIM, OUT_DIM), f32)
    RM = jnp.zeros((OUT_DIM, OUT_DIM), f32)
    for i in range(H):
        RA = RA.at[i * DK:(i + 1) * DK, i * DK:(i + 1) * DK].set(rel_att[i])
        RM = RM.at[i * DK:(i + 1) * DK, i * DK:(i + 1) * DK].set(rel_msg[i])
    pri = (jnp.repeat(rel_pri, DK) * (1.0 / math.sqrt(DK))).reshape(1, OUT_DIM)

    h_pad = jnp.pad(h, ((0, N_PAD - N), (0, 0)))
    q_tab, kv_tab = _qkv(h_pad, WqT, WkT, WvT,
                         bq.reshape(1, -1), bk.reshape(1, -1),
                         bv.reshape(1, -1), RA, RM, pri)

    pad = jnp.full((E_PAD - E,), N, jnp.int32)
    src_pad = jnp.concatenate([edge_index[0], pad])
    dst_pad = jnp.concatenate([edge_index[1], pad])

    agg, den = _sc_edges(q_tab, kv_tab, src_pad, dst_pad)

    col = jnp.arange(OUT_DIM) // DK
    E16 = (col[None, :] == jnp.arange(16)[:, None]).astype(f32)
    alpha = jax.nn.sigmoid(skip[0])
    W2 = Wa.T * alpha
    cb = jnp.zeros((2, 130), f32)
    cb = cb.at[0, 0].set(1.0 - alpha)
    cb = cb.at[1, 1:129].set(ba * alpha)

    return _out(agg[:, :N], den[:, :N], h, E16, W2, cb)


# speedup vs baseline: 30.4385x; 30.4385x over previous
import math

import jax
import jax.numpy as jnp
import numpy as np
from jax import lax
from jax.experimental import pallas as pl
from jax.experimental.pallas import tpu as pltpu
from jax.experimental.pallas import tpu_sc as plsc

N = 10000
E = 320000
IN_DIM = 128
OUT_DIM = 128
H = 8
DK = OUT_DIM // H

NC = 2
NS = 16
HH = H // NC
HW = HH * DK
CHUNK = 128
N_PAD = 10240
E_PER_S = ((E + NS * CHUNK - 1) // (NS * CHUNK)) * CHUNK
E_PAD = E_PER_S * NS
N_CHUNKS = E_PER_S // CHUNK
ROWS_PER_S = N_PAD // NS
ZCHUNKS = ROWS_PER_S // CHUNK

_m = np.arange(HW)
_r = _m % 16
_D_OF = (_m // 16) * 4 + (_r // 4)
_HL_OF = _r % 4
_OLD0 = (_HL_OF + 0 * HH) * DK + _D_OF
_OLD1 = (_HL_OF + 1 * HH) * DK + _D_OF


def _qkv_body(h_ref, wq_ref, wk_ref, wv_ref, bq_ref, bk_ref, bv_ref,
              pq_ref, ra_ref, rm_ref, q_ref, kv_ref):
    hb = h_ref[...]
    q = jnp.dot(hb, wq_ref[...], preferred_element_type=jnp.float32) + bq_ref[...]
    k = jnp.dot(hb, wk_ref[...], preferred_element_type=jnp.float32) + bk_ref[...]
    v = jnp.dot(hb, wv_ref[...], preferred_element_type=jnp.float32) + bv_ref[...]
    qil = jnp.dot(q, pq_ref[...], preferred_element_type=jnp.float32)
    kil = jnp.dot(k, ra_ref[...], preferred_element_type=jnp.float32)
    vil = jnp.dot(v, rm_ref[...], preferred_element_type=jnp.float32)
    for c in range(NC):
        q_ref[c] = qil[:, c * HW:(c + 1) * HW]
        kv_ref[c] = jnp.concatenate(
            [kil[:, c * HW:(c + 1) * HW], vil[:, c * HW:(c + 1) * HW]], axis=1)


def _qkv(h_pad, WqT, WkT, WvT, bq, bk, bv, PQ, RAP, RMP):
    blk = 512
    grid = N_PAD // blk
    full = lambda shape: pl.BlockSpec(shape, lambda i: tuple(0 for _ in shape))
    return pl.pallas_call(
        _qkv_body,
        grid=(grid,),
        in_specs=[
            pl.BlockSpec((blk, IN_DIM), lambda i: (i, 0)),
            full((IN_DIM, OUT_DIM)), full((IN_DIM, OUT_DIM)), full((IN_DIM, OUT_DIM)),
            full((1, OUT_DIM)), full((1, OUT_DIM)), full((1, OUT_DIM)),
            full((OUT_DIM, OUT_DIM)), full((OUT_DIM, OUT_DIM)),
            full((OUT_DIM, OUT_DIM)),
        ],
        out_specs=[
            pl.BlockSpec((NC, blk, HW), lambda i: (0, i, 0)),
            pl.BlockSpec((NC, blk, 2 * HW), lambda i: (0, i, 0)),
        ],
        out_shape=[
            jax.ShapeDtypeStruct((NC, N_PAD, HW), jnp.float32),
            jax.ShapeDtypeStruct((NC, N_PAD, 2 * HW), jnp.float32),
        ],
    )(h_pad, WqT, WkT, WvT, bq, bk, bv, PQ, RAP, RMP)


def _perm16(x, xor_mask):
    idx = (lax.iota(jnp.int32, 16) ^ xor_mask)[:, None]
    dnums = lax.GatherDimensionNumbers(
        offset_dims=(), collapsed_slice_dims=(0,), start_index_map=(0,))
    return lax.gather(x, idx, dnums, (1,),
                      mode=lax.GatherScatterMode.PROMISE_IN_BOUNDS)


def _sc_body(q_hbm, kv_hbm, src_hbm, dst_hbm, agg_hbm, den_hbm,
             sidx, didx, gidx, qbuf, kvbuf, wvbuf, tbuf, agg_sh, den_sh,
             sem_q, sem_kv):
    c = lax.axis_index("c")
    s = lax.axis_index("s")
    base_off = c * N_PAD

    def zrow(i, _):
        for j in range(HW // 16):
            wvbuf[i, pl.ds(j * 16, 16)] = jnp.zeros((16,), jnp.float32)
        tbuf[i, :] = jnp.zeros((16,), jnp.float32)
        return 0
    lax.fori_loop(0, CHUNK, zrow, 0)
    for r in range(ZCHUNKS):
        base = s * ROWS_PER_S + r * CHUNK
        pltpu.sync_copy(wvbuf, agg_sh.at[pl.ds(base, CHUNK)])
        pltpu.sync_copy(tbuf, den_sh.at[pl.ds(base, CHUNK)])
    plsc.subcore_barrier()

    def chunk_body(g, _):
        ebase = s * E_PER_S + g * CHUNK
        pltpu.sync_copy(src_hbm.at[pl.ds(ebase, CHUNK)], sidx)
        pltpu.sync_copy(dst_hbm.at[pl.ds(ebase, CHUNK)], didx)
        for j in range(CHUNK // 16):
            sl = pl.ds(j * 16, 16)
            sidx[sl] = sidx[sl] + base_off
            gidx[sl] = didx[sl] + base_off
        cp_q = pltpu.async_copy(q_hbm.at[gidx], qbuf, sem_q)
        cp_kv = pltpu.async_copy(kv_hbm.at[sidx], kvbuf, sem_kv)
        cp_q.wait()
        cp_kv.wait()

        def edge_body(e, _):
            acc = qbuf[e, pl.ds(0, 16)] * kvbuf[e, pl.ds(0, 16)]
            for j in range(1, HW // 16):
                acc = acc + qbuf[e, pl.ds(j * 16, 16)] * kvbuf[e, pl.ds(j * 16, 16)]
            acc = acc + _perm16(acc, 8)
            acc = acc + _perm16(acc, 4)
            ex16 = jnp.exp(acc)
            tbuf[e, :] = ex16
            for j in range(HW // 16):
                vv = kvbuf[e, pl.ds(HW + j * 16, 16)]
                wvbuf[e, pl.ds(j * 16, 16)] = vv * ex16
            return 0
        lax.fori_loop(0, CHUNK, edge_body, 0)

        pltpu.sync_copy(wvbuf, agg_sh.at[didx], add=True)
        pltpu.sync_copy(tbuf, den_sh.at[didx], add=True)
        return 0
    lax.fori_loop(0, N_CHUNKS, chunk_body, 0)

    plsc.subcore_barrier()
    rbase = s * ROWS_PER_S
    pltpu.sync_copy(agg_sh.at[pl.ds(rbase, ROWS_PER_S)],
                    agg_hbm.at[c, pl.ds(rbase, ROWS_PER_S)])
    pltpu.sync_copy(den_sh.at[pl.ds(rbase, ROWS_PER_S)],
                    den_hbm.at[c, pl.ds(rbase, ROWS_PER_S)])


def _sc_edges(q_tab, kv_tab, src_pad, dst_pad):
    mesh = plsc.VectorSubcoreMesh(core_axis_name="c", subcore_axis_name="s",
                                  num_cores=NC, num_subcores=NS)
    kern = pl.kernel(
        _sc_body,
        out_type=[
            jax.ShapeDtypeStruct((NC, N_PAD, HW), jnp.float32),
            jax.ShapeDtypeStruct((NC, N_PAD, 16), jnp.float32),
        ],
        mesh=mesh,
        compiler_params=pltpu.CompilerParams(use_tc_tiling_on_sc=False),
        scratch_types=[
            pltpu.VMEM((CHUNK,), jnp.int32),
            pltpu.VMEM((CHUNK,), jnp.int32),
            pltpu.VMEM((CHUNK,), jnp.int32),
            pltpu.VMEM((CHUNK, HW), jnp.float32),
            pltpu.VMEM((CHUNK, 2 * HW), jnp.float32),
            pltpu.VMEM((CHUNK, HW), jnp.float32),
            pltpu.VMEM((CHUNK, 16), jnp.float32),
            pltpu.VMEM_SHARED((N_PAD, HW), jnp.float32),
            pltpu.VMEM_SHARED((N_PAD, 16), jnp.float32),
            pltpu.SemaphoreType.DMA,
            pltpu.SemaphoreType.DMA,
        ],
    )
    return kern(q_tab, kv_tab, src_pad, dst_pad)


def _out_body(agg_ref, den_ref, h_ref, e0_ref, e1_ref, w2_ref, wh_ref, b_ref,
              out_ref):
    a = jnp.concatenate([agg_ref[0], agg_ref[1]], axis=1)
    dexp = (jnp.dot(den_ref[0], e0_ref[...], preferred_element_type=jnp.float32)
            + jnp.dot(den_ref[1], e1_ref[...], preferred_element_type=jnp.float32))
    x = a / (dexp + 1e-16)
    t = jnp.dot(x, w2_ref[...], preferred_element_type=jnp.float32)
    t = t + jnp.dot(h_ref[...], wh_ref[...], preferred_element_type=jnp.float32)
    out_ref[...] = t + b_ref[...]


def _out(agg, den, h, E0, E1, W2, Wh, b):
    blk = 1000
    grid = N // blk
    full = lambda shape: pl.BlockSpec(shape, lambda i: tuple(0 for _ in shape))
    return pl.pallas_call(
        _out_body,
        grid=(grid,),
        in_specs=[
            pl.BlockSpec((NC, blk, HW), lambda i: (0, i, 0)),
            pl.BlockSpec((NC, blk, 16), lambda i: (0, i, 0)),
            pl.BlockSpec((blk, IN_DIM), lambda i: (i, 0)),
            full((16, OUT_DIM)),
            full((16, OUT_DIM)),
            full((OUT_DIM, OUT_DIM)),
            full((IN_DIM, OUT_DIM)),
            full((1, OUT_DIM)),
        ],
        out_specs=pl.BlockSpec((blk, OUT_DIM), lambda i: (i, 0)),
        out_shape=jax.ShapeDtypeStruct((N, OUT_DIM), jnp.float32),
    )(agg, den, h, E0, E1, W2, Wh, b)


@jax.jit
def kernel(h, edge_index, Wk, bk, Wq, bq, Wv, bv, Wa, ba, rel_att, rel_msg,
           rel_pri, skip):
    f32 = jnp.float32
    WqT = Wq.T
    WkT = Wk.T
    WvT = Wv.T
    RA = jnp.zeros((OUT_DIM, OUT_DIM), f32)
    RM = jnp.zeros((OUT_DIM, OUT_DIM), f32)
    for i in range(H):
        RA = RA.at[i * DK:(i + 1) * DK, i * DK:(i + 1) * DK].set(rel_att[i])
        RM = RM.at[i * DK:(i + 1) * DK, i * DK:(i + 1) * DK].set(rel_msg[i])
    pri = jnp.repeat(rel_pri, DK) * (1.0 / math.sqrt(DK))

    oldidx = np.concatenate([_OLD0, _OLD1])
    PQ = (jnp.diag(pri))[:, oldidx]
    RAP = RA[:, oldidx]
    RMP = RM[:, oldidx]

    h_pad = jnp.pad(h, ((0, N_PAD - N), (0, 0)))
    q_tab, kv_tab = _qkv(h_pad, WqT, WkT, WvT,
                         bq.reshape(1, -1), bk.reshape(1, -1),
                         bv.reshape(1, -1), PQ, RAP, RMP)

    pad = jnp.full((E_PAD - E,), N, jnp.int32)
    src_pad = jnp.concatenate([edge_index[0], pad])
    dst_pad = jnp.concatenate([edge_index[1], pad])

    agg, den = _sc_edges(q_tab.reshape(NC * N_PAD, HW),
                         kv_tab.reshape(NC * N_PAD, 2 * HW),
                         src_pad, dst_pad)

    nn = np.arange(OUT_DIM)
    hl_of_n = (nn % 16) % 4
    j16 = np.arange(16)
    E0 = ((j16[:, None] == hl_of_n[None, :]) & (j16[:, None] < 4)
          & (nn[None, :] < HW)).astype(np.float32)
    E1 = ((j16[:, None] == hl_of_n[None, :]) & (j16[:, None] < 4)
          & (nn[None, :] >= HW)).astype(np.float32)
    alpha = jax.nn.sigmoid(skip[0])
    W2 = (Wa.T * alpha)[oldidx, :]
    Wh = jnp.eye(IN_DIM, dtype=f32) * (1.0 - alpha)
    b = (ba * alpha).reshape(1, OUT_DIM)

    return _out(agg[:, :N], den[:, :N], h,
                jnp.asarray(E0), jnp.asarray(E1), W2, Wh, b)
